# Initial kernel scaffold; baseline (speedup 1.0000x reference)
import jax
import jax.numpy as jnp
from jax.experimental import pallas as pl


def kernel(inputs, table):
    raise NotImplementedError("write your pallas kernel here")


# speedup vs baseline: 3.8501x; 3.8501x over previous
import functools
import jax
import jax.numpy as jnp
from jax import lax
from jax.experimental import pallas as pl
from jax.experimental.pallas import tpu as pltpu
from jax.experimental.pallas import tpu_sc as plsc

NUM_TYPE = 17
EMB_DIM = 10
ROWS = 16384
COLS = 200
N_IDX = ROWS * COLS
N_OUT = N_IDX * EMB_DIM
NC = 2
NS = 16
NW = NC * NS
PER_W = N_IDX // NW
CHUNK = 2048
N_CHUNK = PER_W // CHUNK
L = 16

_mesh = plsc.VectorSubcoreMesh(core_axis_name="c", subcore_axis_name="s")


@functools.partial(
    pl.kernel,
    mesh=_mesh,
    compiler_params=pltpu.CompilerParams(needs_layout_passes=False),
    out_type=jax.ShapeDtypeStruct((N_OUT,), jnp.float32),
    scratch_types=[
        pltpu.VMEM((NUM_TYPE * EMB_DIM,), jnp.float32),
        pltpu.VMEM((CHUNK,), jnp.int32),
        pltpu.VMEM((CHUNK * EMB_DIM,), jnp.float32),
    ],
)
def _sc_lookup(tab_hbm, idx_hbm, out_hbm, tab_v, idx_v, out_v):
    wid = lax.axis_index("s") * NC + lax.axis_index("c")
    base = wid * PER_W
    pltpu.sync_copy(tab_hbm, tab_v)

    def chunk_body(c, carry):
        start = base + c * CHUNK
        pltpu.sync_copy(idx_hbm.at[pl.ds(start, CHUNK)], idx_v)

        def group_body(g, gcarry):
            g16 = g * L
            lane = lax.broadcasted_iota(jnp.int32, (L,), 0)
            for v in range(EMB_DIM):
                w = lane + (L * v)
                r = w // EMB_DIM
                m = w - r * EMB_DIM
                e = plsc.load_gather(idx_v, [g16 + r])
                a = e * EMB_DIM + m
                wv = plsc.load_gather(tab_v, [a])
                out_v[pl.ds(g * (L * EMB_DIM) + v * L, L)] = wv
            return gcarry

        lax.fori_loop(0, CHUNK // L, group_body, 0)
        pltpu.sync_copy(out_v, out_hbm.at[pl.ds(start * EMB_DIM, CHUNK * EMB_DIM)])
        return carry

    lax.fori_loop(0, N_CHUNK, chunk_body, 0)


def kernel(inputs, table):
    idx = inputs.reshape(-1).astype(jnp.int32)
    tab = table.reshape(-1)
    out = _sc_lookup(tab, idx)
    return out.reshape(ROWS, COLS, EMB_DIM)
